# baseline (speedup 1.0000x reference)
import functools

import jax
import jax.numpy as jnp
from jax import lax
from jax.experimental import pallas as pl
from jax.experimental.pallas import tpu as pltpu
from jax.experimental.pallas import tpu_sc as plsc


def _make_sc_kernel(batch, height, width, num_pos, embed_dim):
    lanes = 16
    n_workers = 32
    ch_per_w = (2 * embed_dim) // n_workers
    kvecs = width // lanes
    mesh = plsc.VectorSubcoreMesh(core_axis_name="c", subcore_axis_name="s")

    @functools.partial(
        pl.kernel,
        mesh=mesh,
        out_type=jax.ShapeDtypeStruct(
            (batch, 2 * embed_dim, height, width), jnp.float32
        ),
        scratch_types=[
            pltpu.VMEM((height, embed_dim), jnp.float32),
            pltpu.VMEM((height, width), jnp.float32),
            pltpu.VMEM((height, width), jnp.float32),
            pltpu.SemaphoreType.DMA((2,)),
        ],
        compiler_params=pltpu.CompilerParams(
            use_tc_tiling_on_sc=False, needs_layout_passes=False
        ),
    )
    def sc_kernel(col_hbm, row_hbm, out_hbm, tab_v, buf_a, buf_b, sems):
        wid = lax.axis_index("s") * 2 + lax.axis_index("c")
        is_x = wid < (n_workers // 2)
        bufs = [buf_a, buf_b]

        @pl.when(is_x)
        def _stage_col():
            pltpu.sync_copy(col_hbm.at[pl.ds(0, width), :], tab_v)

        @pl.when(jnp.logical_not(is_x))
        def _stage_row():
            pltpu.sync_copy(row_hbm.at[pl.ds(0, height), :], tab_v)

        def copies(ci):
            buf = bufs[ci % 2]
            ch = wid * ch_per_w + ci
            return [
                pltpu.make_async_copy(
                    buf, out_hbm.at[b, ch], sems.at[ci % 2]
                )
                for b in range(batch)
            ]

        for ci in range(ch_per_w):
            if ci >= 2:
                for cp in copies(ci - 2):
                    cp.wait()
            buf = bufs[ci % 2]
            ch = wid * ch_per_w + ci

            @pl.when(is_x)
            def _fill_x(buf=buf, ch=ch):
                chv = jnp.full((lanes,), ch, jnp.int32)
                vecs = [
                    plsc.load_gather(
                        tab_v,
                        [lax.iota(jnp.int32, lanes) + k * lanes, chv],
                    )
                    for k in range(kvecs)
                ]

                def body(h, carry):
                    for k in range(kvecs):
                        buf[h, pl.ds(k * lanes, lanes)] = vecs[k]
                    return carry

                lax.fori_loop(0, height, body, 0)

            @pl.when(jnp.logical_not(is_x))
            def _fill_y(buf=buf, ch=ch):
                chv = jnp.full((lanes,), ch - embed_dim, jnp.int32)

                def body(h, carry):
                    v = plsc.load_gather(
                        tab_v, [jnp.full((lanes,), h, jnp.int32), chv]
                    )
                    for k in range(kvecs):
                        buf[h, pl.ds(k * lanes, lanes)] = v
                    return carry

                lax.fori_loop(0, height, body, 0)

            for cp in copies(ci):
                cp.start()

        for ci in range(max(ch_per_w - 2, 0), ch_per_w):
            for cp in copies(ci):
                cp.wait()

    return sc_kernel


def kernel(pixel_values, row_weight, col_weight):
    batch = pixel_values.shape[0]
    height, width = pixel_values.shape[-2], pixel_values.shape[-1]
    num_pos, embed_dim = row_weight.shape
    sc = _make_sc_kernel(batch, height, width, num_pos, embed_dim)
    return sc(col_weight, row_weight)


# speedup vs baseline: 1.5165x; 1.5165x over previous
import functools

import jax
import jax.numpy as jnp
from jax import lax
from jax.experimental import pallas as pl
from jax.experimental.pallas import tpu as pltpu
from jax.experimental.pallas import tpu_sc as plsc


def _make_sc_kernel(batch, height, width, num_pos, embed_dim):
    lanes = 16
    n_workers = 32
    ch_per_w = (2 * embed_dim) // n_workers
    kvecs = width // lanes
    mesh = plsc.VectorSubcoreMesh(core_axis_name="c", subcore_axis_name="s")

    @functools.partial(
        pl.kernel,
        mesh=mesh,
        out_type=jax.ShapeDtypeStruct(
            (batch, 2 * embed_dim, height, width), jnp.float32
        ),
        scratch_types=[
            pltpu.VMEM((height, embed_dim), jnp.float32),
            pltpu.VMEM((height // 2, width), jnp.float32),
            pltpu.VMEM((height // 2, width), jnp.float32),
            pltpu.SemaphoreType.DMA((2,)),
        ],
        compiler_params=pltpu.CompilerParams(
            use_tc_tiling_on_sc=True, needs_layout_passes=False
        ),
    )
    def sc_kernel(col_hbm, row_hbm, out_hbm, tab_v, buf_a, buf_b, sems):
        wid = lax.axis_index("s") * 2 + lax.axis_index("c")
        is_x = wid < (n_workers // 2)
        bufs = [buf_a, buf_b]

        @pl.when(is_x)
        def _stage_col():
            pltpu.sync_copy(col_hbm.at[pl.ds(0, width), :], tab_v)

        @pl.when(jnp.logical_not(is_x))
        def _stage_row():
            pltpu.sync_copy(row_hbm.at[pl.ds(0, height), :], tab_v)

        hh = height // 2
        n_stages = ch_per_w * 2

        def copies(st):
            buf = bufs[st % 2]
            ci, half = st // 2, st % 2
            ch = wid * ch_per_w + ci
            return [
                pltpu.make_async_copy(
                    buf,
                    out_hbm.at[b, ch, pl.ds(half * hh, hh), :],
                    sems.at[st % 2],
                )
                for b in range(batch)
            ]

        for st in range(n_stages):
            if st >= 2:
                for cp in copies(st - 2):
                    cp.wait()
            buf = bufs[st % 2]
            ci, half = st // 2, st % 2
            ch = wid * ch_per_w + ci

            @pl.when(is_x)
            def _fill_x(buf=buf, ch=ch):
                chv = jnp.full((lanes,), ch, jnp.int32)
                vecs = [
                    plsc.load_gather(
                        tab_v,
                        [lax.iota(jnp.int32, lanes) + k * lanes, chv],
                    )
                    for k in range(kvecs)
                ]

                def body(h, carry):
                    for k in range(kvecs):
                        buf[h, pl.ds(k * lanes, lanes)] = vecs[k]
                    return carry

                lax.fori_loop(0, hh, body, 0)

            @pl.when(jnp.logical_not(is_x))
            def _fill_y(buf=buf, ch=ch, half=half):
                chv = jnp.full((lanes,), ch - embed_dim, jnp.int32)

                def body(h, carry):
                    v = plsc.load_gather(
                        tab_v,
                        [jnp.full((lanes,), half * hh, jnp.int32) + h, chv],
                    )
                    for k in range(kvecs):
                        buf[h, pl.ds(k * lanes, lanes)] = v
                    return carry

                lax.fori_loop(0, hh, body, 0)

            for cp in copies(st):
                cp.start()

        for st in range(max(n_stages - 2, 0), n_stages):
            for cp in copies(st):
                cp.wait()

    return sc_kernel


def kernel(pixel_values, row_weight, col_weight):
    batch = pixel_values.shape[0]
    height, width = pixel_values.shape[-2], pixel_values.shape[-1]
    num_pos, embed_dim = row_weight.shape
    sc = _make_sc_kernel(batch, height, width, num_pos, embed_dim)
    return sc(col_weight, row_weight)


# speedup vs baseline: 2.0856x; 1.3752x over previous
import jax
import jax.numpy as jnp
from jax.experimental import pallas as pl
from jax.experimental.pallas import tpu as pltpu

_NBUF = 8
_CBLK = 32


def _pos_kernel(col_ref, row_ref, out_ref, *rest):
    bufs = rest[:_NBUF]
    sems = rest[_NBUF]
    batch = out_ref.shape[0]
    hw = out_ref.shape[2]
    cblk = _CBLK
    embed_dim = col_ref.shape[1]
    n_stages = out_ref.shape[1] // cblk
    nx = embed_dim // cblk
    w = col_ref.shape[0]
    h = row_ref.shape[0]

    xt = col_ref[...].T
    yt = row_ref[...].T

    def copies(k):
        buf = bufs[k % _NBUF]
        return [
            pltpu.make_async_copy(
                buf,
                out_ref.at[b, pl.ds(k * cblk, cblk), :],
                sems.at[k % _NBUF, b],
            )
            for b in range(batch)
        ]

    for k in range(n_stages):
        if k >= _NBUF:
            for c in copies(k - _NBUF):
                c.wait()
        buf = bufs[k % _NBUF]
        if k < nx:
            blk = xt[k * cblk : (k + 1) * cblk, :]
            buf[...] = jnp.broadcast_to(
                blk[:, None, :], (cblk, h, w)
            ).reshape(cblk, hw)
        else:
            blk = yt[(k - nx) * cblk : (k - nx + 1) * cblk, :]
            buf[...] = jnp.broadcast_to(
                blk[:, :, None], (cblk, h, w)
            ).reshape(cblk, hw)
        for c in copies(k):
            c.start()

    for k in range(max(n_stages - _NBUF, 0), n_stages):
        for c in copies(k):
            c.wait()


def kernel(pixel_values, row_weight, col_weight):
    batch = pixel_values.shape[0]
    height, width = pixel_values.shape[-2], pixel_values.shape[-1]
    embed_dim = row_weight.shape[1]

    out = pl.pallas_call(
        _pos_kernel,
        in_specs=[
            pl.BlockSpec(memory_space=pltpu.MemorySpace.VMEM),
            pl.BlockSpec(memory_space=pltpu.MemorySpace.VMEM),
        ],
        out_specs=pl.BlockSpec(memory_space=pltpu.MemorySpace.HBM),
        out_shape=jax.ShapeDtypeStruct(
            (batch, 2 * embed_dim, height * width), jnp.float32
        ),
        scratch_shapes=[
            pltpu.VMEM((_CBLK, height * width), jnp.float32)
            for _ in range(_NBUF)
        ]
        + [pltpu.SemaphoreType.DMA((_NBUF, batch))],
    )(col_weight[:width, :], row_weight[:height, :])
    return out.reshape(batch, 2 * embed_dim, height, width)
